# baseline (speedup 1.0000x reference)
import functools

import jax
import jax.numpy as jnp
from jax import lax
from jax.experimental import pallas as pl
from jax.experimental.pallas import tpu as pltpu
from jax.experimental.pallas import tpu_sc as plsc

N = 10000
E = 320000
D = 128
R = 8
G = 64

NC = 2
NS = 16
NW = NC * NS

CHB = 128
NCH = 80
EP = NW * NCH * CHB
EROWS = EP // CHB
CROWS = EROWS // NS
NRB = N * R
CNTSZ = 80128
ZPT = CNTSZ // NS
RPT = 624

_mesh = plsc.VectorSubcoreMesh(core_axis_name="c", subcore_axis_name="s")
_sc_params = pltpu.CompilerParams(needs_layout_passes=False)


@functools.partial(
    pl.kernel,
    out_type=jax.ShapeDtypeStruct((EROWS, CHB), jnp.float32),
    mesh=_mesh,
    compiler_params=_sc_params,
    scratch_types=[
        pltpu.VMEM((CROWS, CHB), jnp.int32),
        pltpu.VMEM((CHB,), jnp.float32),
        pltpu.VMEM((CHB,), jnp.float32),
        pltpu.VMEM((CNTSZ,), jnp.float32),
        pltpu.VMEM((NCH, CHB), jnp.float32),
        pltpu.VMEM_SHARED((CNTSZ,), jnp.float32),
        pltpu.SemaphoreType.DMA,
    ],
)
def _sc_scales(fdst_hbm, scale_hbm, fdst_v, ones_v, zb, cnt_loc, scale_loc,
               cnt_sh, cnt_sem):
  c = lax.axis_index("c")
  t = lax.axis_index("s")

  one = jnp.ones((16,), jnp.float32)
  zero = jnp.zeros((16,), jnp.float32)
  for j in range(CHB // 16):
    ones_v[pl.ds(j * 16, 16)] = one
    zb[pl.ds(j * 16, 16)] = zero

  z0 = t * ZPT
  @pl.loop(0, ZPT // CHB)
  def _(j):
    pltpu.sync_copy(zb, cnt_sh.at[pl.ds(z0 + j * CHB, CHB)])
  zrem = ZPT - (ZPT // CHB) * CHB
  pltpu.sync_copy(zb.at[pl.ds(0, zrem)],
                  cnt_sh.at[pl.ds(z0 + ZPT - zrem, zrem)])
  plsc.subcore_barrier()

  pltpu.sync_copy(fdst_hbm.at[pl.ds(t * CROWS, CROWS)], fdst_v)
  @pl.loop(0, CROWS)
  def _(k):
    pltpu.async_copy(ones_v, cnt_sh.at[fdst_v.at[k]], cnt_sem, add=True)
    @pl.when(k >= 8)
    def _():
      pltpu.make_async_copy(scale_hbm.at[0], ones_v, cnt_sem).wait()
  for _ in range(8):
    pltpu.make_async_copy(scale_hbm.at[0], ones_v, cnt_sem).wait()
  plsc.subcore_barrier()

  w = c * NS + t
  pltpu.sync_copy(fdst_hbm.at[pl.ds(w * NCH, NCH)], fdst_v.at[pl.ds(0, NCH)])
  pltpu.sync_copy(cnt_sh, cnt_loc)
  @pl.loop(0, NCH)
  def _(row):
    for j in range(CHB // 16):
      v = fdst_v[row, pl.ds(j * 16, 16)]
      cv = plsc.load_gather(cnt_loc, [v])
      s = jnp.where(v < NRB, 1.0 / jnp.maximum(cv, 1.0), 0.0)
      scale_loc[row, pl.ds(j * 16, 16)] = s
  pltpu.sync_copy(scale_loc, scale_hbm.at[pl.ds(w * NCH, NCH)])


@functools.partial(
    pl.kernel,
    out_type=jax.ShapeDtypeStruct((NC, N, D), jnp.float32),
    mesh=_mesh,
    compiler_params=_sc_params,
    scratch_types=[
        pltpu.VMEM((4, CHB), jnp.int32),
        pltpu.VMEM((4, CHB), jnp.int32),
        pltpu.VMEM((4, CHB), jnp.float32),
        pltpu.VMEM((3, CHB, D), jnp.float32),
        pltpu.VMEM_SHARED((N, D), jnp.float32),
        pltpu.SemaphoreType.DMA,
        pltpu.SemaphoreType.DMA,
        pltpu.SemaphoreType.DMA,
        pltpu.SemaphoreType.DMA,
        pltpu.SemaphoreType.DMA,
        pltpu.SemaphoreType.DMA,
        pltpu.SemaphoreType.DMA,
        pltpu.SemaphoreType.DMA,
        pltpu.SemaphoreType.DMA,
        pltpu.SemaphoreType.DMA,
        pltpu.SemaphoreType.DMA,
        pltpu.SemaphoreType.DMA,
        pltpu.SemaphoreType.DMA,
    ],
)
def _sc_agg(y_hbm, gidx_hbm, scale_hbm, dst_hbm, out_hbm,
            idx4, dst4, scl4, rows, acc_sh,
            g_sem0, g_sem1, g_sem2, w_sem0, w_sem1, w_sem2,
            l_sem0, l_sem1, l_sem2, l_sem3,
            h_sem0, h_sem1, h_sem2):
  c = lax.axis_index("c")
  t = lax.axis_index("s")
  w = c * NS + t
  g_sems = (g_sem0, g_sem1, g_sem2)
  w_sems = (w_sem0, w_sem1, w_sem2)
  l_sems = (l_sem0, l_sem1, l_sem2, l_sem3)
  h_sems = (h_sem0, h_sem1, h_sem2)

  zero = jnp.zeros((16,), jnp.float32)
  @pl.loop(0, CHB)
  def _(i):
    for j in range(D // 16):
      rows[0, i, pl.ds(j * 16, 16)] = zero
  r0 = t * RPT
  for k in range(RPT // CHB):
    pltpu.sync_copy(rows.at[0], acc_sh.at[pl.ds(r0 + k * CHB, CHB)])
  zrem = RPT - (RPT // CHB) * CHB
  pltpu.sync_copy(rows.at[0, pl.ds(0, zrem)],
                  acc_sh.at[pl.ds(r0 + RPT - zrem, zrem)])
  @pl.when(t == NS - 1)
  def _():
    pltpu.sync_copy(rows.at[0, pl.ds(0, N - NS * RPT)],
                    acc_sh.at[pl.ds(NS * RPT, N - NS * RPT)])
  plsc.subcore_barrier()

  ebase = w * NCH

  def fire_loads(u, q):
    pltpu.async_copy(gidx_hbm.at[ebase + q], idx4.at[u], l_sems[u])
    pltpu.async_copy(dst_hbm.at[ebase + q], dst4.at[u], l_sems[u])
    pltpu.async_copy(scale_hbm.at[ebase + q], scl4.at[u], l_sems[u])

  def wait_loads(u):
    pltpu.make_async_copy(gidx_hbm.at[0], idx4.at[u], l_sems[u]).wait()
    pltpu.make_async_copy(dst_hbm.at[0], dst4.at[u], l_sems[u]).wait()
    pltpu.make_async_copy(scale_hbm.at[0], scl4.at[u], l_sems[u]).wait()

  H = CHB // 2

  def fire_gather(u, s):
    pltpu.async_copy(y_hbm.at[idx4.at[u, pl.ds(0, H)]],
                     rows.at[s, pl.ds(0, H)], g_sems[s])
    pltpu.async_copy(y_hbm.at[idx4.at[u, pl.ds(H, H)]],
                     rows.at[s, pl.ds(H, H)], h_sems[s])

  def wait_gather(s):
    pltpu.make_async_copy(y_hbm.at[pl.ds(0, H)], rows.at[s, pl.ds(0, H)],
                          g_sems[s]).wait()
    pltpu.make_async_copy(y_hbm.at[pl.ds(0, H)], rows.at[s, pl.ds(H, H)],
                          h_sems[s]).wait()

  def fire_scatter(s, u):
    pltpu.async_copy(rows.at[s], acc_sh.at[dst4.at[u]], w_sems[s],
                     add=True)

  def wait_scatter(s):
    pltpu.make_async_copy(rows.at[s], acc_sh.at[pl.ds(0, CHB)],
                          w_sems[s]).wait()

  def scale_rows(s, u):
    @pl.loop(0, CHB // 16)
    def _(grp):
      sv = scl4[u, pl.ds(grp * 16, 16)]
      for lane in range(16):
        sc = sv[lane]
        row = grp * 16 + lane
        for j in range(D // 16):
          rows[s, row, pl.ds(j * 16, 16)] = (
              rows[s, row, pl.ds(j * 16, 16)] * sc)

  fire_loads(0, 0)
  fire_loads(1, 1)
  fire_loads(2, 2)
  wait_loads(0)
  fire_gather(0, 0)
  wait_loads(1)
  fire_gather(1, 1)

  @pl.loop(0, NCH // 12)
  def _(m):
    q0 = m * 12
    for i in range(12):
      q = q0 + i
      s, u = i % 3, i % 4
      wait_gather(s)
      scale_rows(s, u)
      fire_scatter(s, u)
      if i == 0:
        @pl.when(q >= 1)
        def _():
          wait_scatter((s + 2) % 3)
      else:
        wait_scatter((s + 2) % 3)
      @pl.when(q + 3 < NCH)
      def _():
        fire_loads((u + 3) % 4, q + 3)
      @pl.when(q + 2 < NCH)
      def _():
        wait_loads((u + 2) % 4)
        fire_gather((u + 2) % 4, (s + 2) % 3)
  for q in range(12 * (NCH // 12), NCH):
    s, u = q % 3, q % 4
    wait_gather(s)
    scale_rows(s, u)
    fire_scatter(s, u)
    wait_scatter((s + 2) % 3)
    if q + 3 < NCH:
      fire_loads((u + 3) % 4, q + 3)
    if q + 2 < NCH:
      wait_loads((u + 2) % 4)
      fire_gather((u + 2) % 4, (s + 2) % 3)
  wait_scatter((NCH - 1) % 3)
  plsc.subcore_barrier()

  for k in range(RPT // CHB):
    pltpu.sync_copy(acc_sh.at[pl.ds(r0 + k * CHB, CHB)], rows.at[0])
    pltpu.sync_copy(rows.at[0], out_hbm.at[c, pl.ds(r0 + k * CHB, CHB)])
  pltpu.sync_copy(acc_sh.at[pl.ds(r0 + RPT - zrem, zrem)],
                  rows.at[0, pl.ds(0, zrem)])
  pltpu.sync_copy(rows.at[0, pl.ds(0, zrem)],
                  out_hbm.at[c, pl.ds(r0 + RPT - zrem, zrem)])
  @pl.when(t == NS - 1)
  def _():
    pltpu.sync_copy(acc_sh.at[pl.ds(NS * RPT, N - NS * RPT)],
                    rows.at[0, pl.ds(0, N - NS * RPT)])
    pltpu.sync_copy(rows.at[0, pl.ds(0, N - NS * RPT)],
                    out_hbm.at[c, pl.ds(NS * RPT, N - NS * RPT)])


_TB = 2000


def _t1_body(h_ref, w_ref, b_ref, out0_ref, y_ref):
  res = jnp.dot(h_ref[...], w_ref[...], preferred_element_type=jnp.float32)
  out0_ref[...] = res[:, :D] + b_ref[...]
  y_ref[...] = res[:, D:].reshape(_TB * R, D)


def _t2_body(o_ref, p0_ref, p1_ref, w_ref, b_ref, out0_ref, y_ref):
  h = jax.nn.relu(o_ref[...] + p0_ref[...] + p1_ref[...])
  res = jnp.dot(h, w_ref[...], preferred_element_type=jnp.float32)
  out0_ref[...] = res[:, :D] + b_ref[...]
  y_ref[...] = res[:, D:].reshape(_TB * R, D)


def _t3_body(o_ref, p0_ref, p1_ref, batch_ref, lw_ref, lb_ref, out_ref):
  h = jax.nn.relu(o_ref[...] + p0_ref[...] + p1_ref[...])
  gid = lax.broadcasted_iota(jnp.int32, (N, G), 1)
  eq = (batch_ref[...] == gid).astype(jnp.float32)
  dn = (((0,), (0,)), ((), ()))
  gs = lax.dot_general(eq, h, dn, preferred_element_type=jnp.float32)
  ones = jnp.ones((N, D), jnp.float32)
  cnt = lax.dot_general(eq, ones, dn, preferred_element_type=jnp.float32)
  g = gs / jnp.maximum(cnt, 1.0)
  out_ref[...] = jnp.dot(g, lw_ref[...], preferred_element_type=jnp.float32) + lb_ref[...]


def _tc_layer1(h, wcat, b):
  grid = N // _TB
  return pl.pallas_call(
      _t1_body,
      grid=(grid,),
      in_specs=[
          pl.BlockSpec((_TB, D), lambda i: (i, 0)),
          pl.BlockSpec((D, D * (R + 1)), lambda i: (0, 0)),
          pl.BlockSpec((1, D), lambda i: (0, 0)),
      ],
      out_specs=[
          pl.BlockSpec((_TB, D), lambda i: (i, 0)),
          pl.BlockSpec((_TB * R, D), lambda i: (i, 0)),
      ],
      out_shape=[
          jax.ShapeDtypeStruct((N, D), jnp.float32),
          jax.ShapeDtypeStruct((N * R, D), jnp.float32),
      ],
  )(h, wcat, b)


def _tc_layer2(o, p0, p1, wcat, b):
  grid = N // _TB
  return pl.pallas_call(
      _t2_body,
      grid=(grid,),
      in_specs=[
          pl.BlockSpec((_TB, D), lambda i: (i, 0)),
          pl.BlockSpec((_TB, D), lambda i: (i, 0)),
          pl.BlockSpec((_TB, D), lambda i: (i, 0)),
          pl.BlockSpec((D, D * (R + 1)), lambda i: (0, 0)),
          pl.BlockSpec((1, D), lambda i: (0, 0)),
      ],
      out_specs=[
          pl.BlockSpec((_TB, D), lambda i: (i, 0)),
          pl.BlockSpec((_TB * R, D), lambda i: (i, 0)),
      ],
      out_shape=[
          jax.ShapeDtypeStruct((N, D), jnp.float32),
          jax.ShapeDtypeStruct((N * R, D), jnp.float32),
      ],
  )(o, p0, p1, wcat, b)


def _tc_final(o, p0, p1, batch2d, lw, lb):
  return pl.pallas_call(
      _t3_body,
      out_shape=jax.ShapeDtypeStruct((G, D), jnp.float32),
  )(o, p0, p1, batch2d, lw, lb)


def _wcat(w_root, w_rel):
  return jnp.concatenate(
      [w_root, w_rel.transpose(1, 0, 2).reshape(D, R * D)], axis=1)


def kernel(x, edge_index, edge_type, batch, W1_root, W1_rel, b1,
           W2_root, W2_rel, b2, lin_W, lin_b):
  src = edge_index[0].astype(jnp.int32)
  dst = edge_index[1].astype(jnp.int32)
  et = edge_type.astype(jnp.int32)
  gidx = src * R + et
  fdst = dst * R + et

  pad = EP - E
  gidx_p = jnp.concatenate([gidx, jnp.zeros((pad,), jnp.int32)])
  dst_p = jnp.concatenate([dst, jnp.zeros((pad,), jnp.int32)])
  fdst_p = jnp.concatenate([fdst, jnp.full((pad,), NRB, jnp.int32)])
  gidx2d = gidx_p.reshape(EROWS, CHB)
  dst2d = dst_p.reshape(EROWS, CHB)
  fdst2d = fdst_p.reshape(EROWS, CHB)

  scale2d = _sc_scales(fdst2d)

  o1, y1 = _tc_layer1(x, _wcat(W1_root, W1_rel), b1.reshape(1, D))
  p1 = _sc_agg(y1, gidx2d, scale2d, dst2d)
  o2, y2 = _tc_layer2(o1, p1[0], p1[1], _wcat(W2_root, W2_rel),
                      b2.reshape(1, D))
  p2 = _sc_agg(y2, gidx2d, scale2d, dst2d)
  out = _tc_final(o2, p2[0], p2[1], batch.astype(jnp.int32).reshape(N, 1),
                  lin_W, lin_b.reshape(1, D))
  return out


# speedup vs baseline: 1.0046x; 1.0046x over previous
import functools

import jax
import jax.numpy as jnp
from jax import lax
from jax.experimental import pallas as pl
from jax.experimental.pallas import tpu as pltpu
from jax.experimental.pallas import tpu_sc as plsc

N = 10000
E = 320000
D = 128
R = 8
G = 64

NC = 2
NS = 16
NW = NC * NS

CHB = 128
NCH = 80
EP = NW * NCH * CHB
EROWS = EP // CHB
CROWS = EROWS // NS
NRB = N * R
CNTSZ = 80128
ZPT = CNTSZ // NS
RPT = 624

_mesh = plsc.VectorSubcoreMesh(core_axis_name="c", subcore_axis_name="s")
_sc_params = pltpu.CompilerParams(needs_layout_passes=False)


@functools.partial(
    pl.kernel,
    out_type=jax.ShapeDtypeStruct((EROWS, CHB), jnp.float32),
    mesh=_mesh,
    compiler_params=_sc_params,
    scratch_types=[
        pltpu.VMEM((CROWS, CHB), jnp.int32),
        pltpu.VMEM((CHB,), jnp.float32),
        pltpu.VMEM((CHB,), jnp.float32),
        pltpu.VMEM((CNTSZ,), jnp.float32),
        pltpu.VMEM((NCH, CHB), jnp.float32),
        pltpu.VMEM_SHARED((CNTSZ,), jnp.float32),
        pltpu.SemaphoreType.DMA,
    ],
)
def _sc_scales(fdst_hbm, scale_hbm, fdst_v, ones_v, zb, cnt_loc, scale_loc,
               cnt_sh, cnt_sem):
  c = lax.axis_index("c")
  t = lax.axis_index("s")

  one = jnp.ones((16,), jnp.float32)
  zero = jnp.zeros((16,), jnp.float32)
  for j in range(CHB // 16):
    ones_v[pl.ds(j * 16, 16)] = one
    zb[pl.ds(j * 16, 16)] = zero

  z0 = t * ZPT
  @pl.loop(0, ZPT // CHB)
  def _(j):
    pltpu.sync_copy(zb, cnt_sh.at[pl.ds(z0 + j * CHB, CHB)])
  zrem = ZPT - (ZPT // CHB) * CHB
  pltpu.sync_copy(zb.at[pl.ds(0, zrem)],
                  cnt_sh.at[pl.ds(z0 + ZPT - zrem, zrem)])
  plsc.subcore_barrier()

  pltpu.sync_copy(fdst_hbm.at[pl.ds(t * CROWS, CROWS)], fdst_v)
  @pl.loop(0, CROWS)
  def _(k):
    pltpu.async_copy(ones_v, cnt_sh.at[fdst_v.at[k]], cnt_sem, add=True)
    @pl.when(k >= 8)
    def _():
      pltpu.make_async_copy(scale_hbm.at[0], ones_v, cnt_sem).wait()
  for _ in range(8):
    pltpu.make_async_copy(scale_hbm.at[0], ones_v, cnt_sem).wait()
  plsc.subcore_barrier()

  w = c * NS + t
  pltpu.sync_copy(fdst_hbm.at[pl.ds(w * NCH, NCH)], fdst_v.at[pl.ds(0, NCH)])
  pltpu.sync_copy(cnt_sh, cnt_loc)
  @pl.loop(0, NCH)
  def _(row):
    for j in range(CHB // 16):
      v = fdst_v[row, pl.ds(j * 16, 16)]
      cv = plsc.load_gather(cnt_loc, [v])
      s = jnp.where(v < NRB, 1.0 / jnp.maximum(cv, 1.0), 0.0)
      scale_loc[row, pl.ds(j * 16, 16)] = s
  pltpu.sync_copy(scale_loc, scale_hbm.at[pl.ds(w * NCH, NCH)])


@functools.partial(
    pl.kernel,
    out_type=jax.ShapeDtypeStruct((NC, N, D), jnp.float32),
    mesh=_mesh,
    compiler_params=_sc_params,
    scratch_types=[
        pltpu.VMEM((4, CHB), jnp.int32),
        pltpu.VMEM((4, CHB), jnp.int32),
        pltpu.VMEM((4, CHB), jnp.float32),
        pltpu.VMEM((3, CHB, D), jnp.float32),
        pltpu.VMEM_SHARED((N, D), jnp.float32),
        pltpu.SemaphoreType.DMA,
        pltpu.SemaphoreType.DMA,
        pltpu.SemaphoreType.DMA,
        pltpu.SemaphoreType.DMA,
        pltpu.SemaphoreType.DMA,
        pltpu.SemaphoreType.DMA,
        pltpu.SemaphoreType.DMA,
        pltpu.SemaphoreType.DMA,
        pltpu.SemaphoreType.DMA,
        pltpu.SemaphoreType.DMA,
    ],
)
def _sc_agg(y_hbm, gidx_hbm, scale_hbm, dst_hbm, out_hbm,
            idx4, dst4, scl4, rows, acc_sh,
            g_sem0, g_sem1, g_sem2, w_sem0, w_sem1, w_sem2,
            l_sem0, l_sem1, l_sem2, l_sem3):
  c = lax.axis_index("c")
  t = lax.axis_index("s")
  w = c * NS + t
  g_sems = (g_sem0, g_sem1, g_sem2)
  w_sems = (w_sem0, w_sem1, w_sem2)
  l_sems = (l_sem0, l_sem1, l_sem2, l_sem3)

  r0 = t * RPT
  zrem = RPT - (RPT // CHB) * CHB
  ebase = w * NCH

  def fire_loads(u, q):
    pltpu.async_copy(gidx_hbm.at[ebase + q], idx4.at[u], l_sems[u])
    pltpu.async_copy(dst_hbm.at[ebase + q], dst4.at[u], l_sems[u])
    pltpu.async_copy(scale_hbm.at[ebase + q], scl4.at[u], l_sems[u])

  def wait_loads(u):
    pltpu.make_async_copy(gidx_hbm.at[0], idx4.at[u], l_sems[u]).wait()
    pltpu.make_async_copy(dst_hbm.at[0], dst4.at[u], l_sems[u]).wait()
    pltpu.make_async_copy(scale_hbm.at[0], scl4.at[u], l_sems[u]).wait()

  def fire_gather(u, s):
    pltpu.async_copy(y_hbm.at[idx4.at[u]], rows.at[s], g_sems[s])

  def wait_gather(s):
    pltpu.make_async_copy(y_hbm.at[pl.ds(0, CHB)], rows.at[s],
                          g_sems[s]).wait()

  def fire_scatter(s, u):
    pltpu.async_copy(rows.at[s], acc_sh.at[dst4.at[u]], w_sems[s],
                     add=True)

  def wait_scatter(s):
    pltpu.make_async_copy(rows.at[s], acc_sh.at[pl.ds(0, CHB)],
                          w_sems[s]).wait()

  def scale_rows(s, u):
    @pl.loop(0, CHB // 16)
    def _(grp):
      sv = scl4[u, pl.ds(grp * 16, 16)]
      for lane in range(16):
        sc = sv[lane]
        row = grp * 16 + lane
        for j in range(D // 16):
          rows[s, row, pl.ds(j * 16, 16)] = (
              rows[s, row, pl.ds(j * 16, 16)] * sc)

  fire_loads(0, 0)
  fire_loads(1, 1)
  fire_loads(2, 2)
  wait_loads(0)
  fire_gather(0, 0)
  wait_loads(1)
  fire_gather(1, 1)

  zero = jnp.zeros((16,), jnp.float32)
  @pl.loop(0, CHB)
  def _(i):
    for j in range(D // 16):
      rows[2, i, pl.ds(j * 16, 16)] = zero
  for k in range(RPT // CHB):
    pltpu.sync_copy(rows.at[2], acc_sh.at[pl.ds(r0 + k * CHB, CHB)])
  pltpu.sync_copy(rows.at[2, pl.ds(0, zrem)],
                  acc_sh.at[pl.ds(r0 + RPT - zrem, zrem)])
  @pl.when(t == NS - 1)
  def _():
    pltpu.sync_copy(rows.at[2, pl.ds(0, N - NS * RPT)],
                    acc_sh.at[pl.ds(NS * RPT, N - NS * RPT)])
  plsc.subcore_barrier()

  @pl.loop(0, NCH // 12)
  def _(m):
    q0 = m * 12
    for i in range(12):
      q = q0 + i
      s, u = i % 3, i % 4
      wait_gather(s)
      scale_rows(s, u)
      fire_scatter(s, u)
      if i == 0:
        @pl.when(q >= 1)
        def _():
          wait_scatter((s + 2) % 3)
      else:
        wait_scatter((s + 2) % 3)
      @pl.when(q + 3 < NCH)
      def _():
        fire_loads((u + 3) % 4, q + 3)
      @pl.when(q + 2 < NCH)
      def _():
        wait_loads((u + 2) % 4)
        fire_gather((u + 2) % 4, (s + 2) % 3)
  for q in range(12 * (NCH // 12), NCH):
    s, u = q % 3, q % 4
    wait_gather(s)
    scale_rows(s, u)
    fire_scatter(s, u)
    wait_scatter((s + 2) % 3)
    if q + 3 < NCH:
      fire_loads((u + 3) % 4, q + 3)
    if q + 2 < NCH:
      wait_loads((u + 2) % 4)
      fire_gather((u + 2) % 4, (s + 2) % 3)
  wait_scatter((NCH - 1) % 3)
  plsc.subcore_barrier()

  pltpu.sync_copy(acc_sh.at[pl.ds(r0, RPT)], out_hbm.at[c, pl.ds(r0, RPT)])
  @pl.when(t == NS - 1)
  def _():
    pltpu.sync_copy(acc_sh.at[pl.ds(NS * RPT, N - NS * RPT)],
                    out_hbm.at[c, pl.ds(NS * RPT, N - NS * RPT)])


_TB = 2000


def _t1_body(h_ref, w_ref, b_ref, out0_ref, y_ref):
  res = jnp.dot(h_ref[...], w_ref[...], preferred_element_type=jnp.float32)
  out0_ref[...] = res[:, :D] + b_ref[...]
  y_ref[...] = res[:, D:].reshape(_TB * R, D)


def _t2_body(o_ref, p0_ref, p1_ref, w_ref, b_ref, out0_ref, y_ref):
  h = jax.nn.relu(o_ref[...] + p0_ref[...] + p1_ref[...])
  res = jnp.dot(h, w_ref[...], preferred_element_type=jnp.float32)
  out0_ref[...] = res[:, :D] + b_ref[...]
  y_ref[...] = res[:, D:].reshape(_TB * R, D)


def _t3_body(o_ref, p0_ref, p1_ref, batch_ref, lw_ref, lb_ref, out_ref):
  h = jax.nn.relu(o_ref[...] + p0_ref[...] + p1_ref[...])
  gid = lax.broadcasted_iota(jnp.int32, (N, G), 1)
  eq = (batch_ref[...] == gid).astype(jnp.float32)
  dn = (((0,), (0,)), ((), ()))
  gs = lax.dot_general(eq, h, dn, preferred_element_type=jnp.float32)
  ones = jnp.ones((N, D), jnp.float32)
  cnt = lax.dot_general(eq, ones, dn, preferred_element_type=jnp.float32)
  g = gs / jnp.maximum(cnt, 1.0)
  out_ref[...] = jnp.dot(g, lw_ref[...], preferred_element_type=jnp.float32) + lb_ref[...]


def _tc_layer1(h, wcat, b):
  grid = N // _TB
  return pl.pallas_call(
      _t1_body,
      grid=(grid,),
      in_specs=[
          pl.BlockSpec((_TB, D), lambda i: (i, 0)),
          pl.BlockSpec((D, D * (R + 1)), lambda i: (0, 0)),
          pl.BlockSpec((1, D), lambda i: (0, 0)),
      ],
      out_specs=[
          pl.BlockSpec((_TB, D), lambda i: (i, 0)),
          pl.BlockSpec((_TB * R, D), lambda i: (i, 0)),
      ],
      out_shape=[
          jax.ShapeDtypeStruct((N, D), jnp.float32),
          jax.ShapeDtypeStruct((N * R, D), jnp.float32),
      ],
  )(h, wcat, b)


def _tc_layer2(o, p0, p1, wcat, b):
  grid = N // _TB
  return pl.pallas_call(
      _t2_body,
      grid=(grid,),
      in_specs=[
          pl.BlockSpec((_TB, D), lambda i: (i, 0)),
          pl.BlockSpec((_TB, D), lambda i: (i, 0)),
          pl.BlockSpec((_TB, D), lambda i: (i, 0)),
          pl.BlockSpec((D, D * (R + 1)), lambda i: (0, 0)),
          pl.BlockSpec((1, D), lambda i: (0, 0)),
      ],
      out_specs=[
          pl.BlockSpec((_TB, D), lambda i: (i, 0)),
          pl.BlockSpec((_TB * R, D), lambda i: (i, 0)),
      ],
      out_shape=[
          jax.ShapeDtypeStruct((N, D), jnp.float32),
          jax.ShapeDtypeStruct((N * R, D), jnp.float32),
      ],
  )(o, p0, p1, wcat, b)


def _tc_final(o, p0, p1, batch2d, lw, lb):
  return pl.pallas_call(
      _t3_body,
      out_shape=jax.ShapeDtypeStruct((G, D), jnp.float32),
  )(o, p0, p1, batch2d, lw, lb)


def _wcat(w_root, w_rel):
  return jnp.concatenate(
      [w_root, w_rel.transpose(1, 0, 2).reshape(D, R * D)], axis=1)


def kernel(x, edge_index, edge_type, batch, W1_root, W1_rel, b1,
           W2_root, W2_rel, b2, lin_W, lin_b):
  src = edge_index[0].astype(jnp.int32)
  dst = edge_index[1].astype(jnp.int32)
  et = edge_type.astype(jnp.int32)
  gidx = src * R + et
  fdst = dst * R + et

  pad = EP - E
  gidx_p = jnp.concatenate([gidx, jnp.zeros((pad,), jnp.int32)])
  dst_p = jnp.concatenate([dst, jnp.zeros((pad,), jnp.int32)])
  fdst_p = jnp.concatenate([fdst, jnp.full((pad,), NRB, jnp.int32)])
  gidx2d = gidx_p.reshape(EROWS, CHB)
  dst2d = dst_p.reshape(EROWS, CHB)
  fdst2d = fdst_p.reshape(EROWS, CHB)

  scale2d = _sc_scales(fdst2d)

  o1, y1 = _tc_layer1(x, _wcat(W1_root, W1_rel), b1.reshape(1, D))
  p1 = _sc_agg(y1, gidx2d, scale2d, dst2d)
  o2, y2 = _tc_layer2(o1, p1[0], p1[1], _wcat(W2_root, W2_rel),
                      b2.reshape(1, D))
  p2 = _sc_agg(y2, gidx2d, scale2d, dst2d)
  out = _tc_final(o2, p2[0], p2[1], batch.astype(jnp.int32).reshape(N, 1),
                  lin_W, lin_b.reshape(1, D))
  return out
